# Initial kernel scaffold; baseline (speedup 1.0000x reference)
import functools

import jax
import jax.numpy as jnp
from jax import lax
from jax.experimental import pallas as pl
from jax.experimental.pallas import tpu as pltpu
from jax.experimental.pallas import tpu_sc as plsc

N = 10000
NPAD = 10016
K = 128
NW = 32
NSUB = 16
RPT = NPAD // NSUB


def _scatter_body(ept, nch, h_hbm, src_hbm, dst_hbm, zero_hbm, out_hbm,
                  acc_sh, sidx, didx, rows, sem):
    cid = lax.axis_index("c")
    sid = lax.axis_index("s")
    wid = sid * 2 + cid

    @pl.when(sid == 0)
    def _zero():
        pltpu.sync_copy(zero_hbm, acc_sh)

    plsc.subcore_barrier()

    base = wid * ept

    def chunk(g, carry):
        eb = pl.multiple_of(base + g * K, 8)
        pltpu.sync_copy(src_hbm.at[pl.ds(eb, K)], sidx)
        pltpu.sync_copy(dst_hbm.at[pl.ds(eb, K)], didx)
        pltpu.async_copy(h_hbm.at[sidx], rows, sem).wait()
        pltpu.sync_copy(rows, acc_sh.at[didx], add=True)
        return carry

    lax.fori_loop(0, nch, chunk, 0)
    plsc.subcore_barrier()
    row0 = cid * NPAD + sid * RPT
    pltpu.sync_copy(acc_sh.at[pl.ds(sid * RPT, RPT)],
                    out_hbm.at[pl.ds(row0, RPT)])


@functools.lru_cache(maxsize=None)
def _make_scatter(c, ept, nch):
    mesh = plsc.VectorSubcoreMesh(core_axis_name="c", subcore_axis_name="s")
    return pl.kernel(
        functools.partial(_scatter_body, ept, nch),
        out_type=jax.ShapeDtypeStruct((2 * NPAD, c), jnp.float32),
        mesh=mesh,
        scratch_types=[
            pltpu.VMEM_SHARED((NPAD, c), jnp.float32),
            pltpu.VMEM((K,), jnp.int32),
            pltpu.VMEM((K,), jnp.int32),
            pltpu.VMEM((K, c), jnp.float32),
            pltpu.SemaphoreType.DMA,
        ],
    )


def _dinv(deg_ref):
    deg = deg_ref[:NPAD, 0:1] + deg_ref[NPAD:, 0:1] + 1.0
    row = lax.broadcasted_iota(jnp.int32, (NPAD, 1), 0)
    return jnp.where(row < N, lax.rsqrt(deg), 0.0)


def _tc1_body(deg_ref, x_ref, w1_ref, h1p_ref):
    h = jnp.dot(x_ref[...], w1_ref[...], preferred_element_type=jnp.float32)
    h1p_ref[...] = _dinv(deg_ref) * h


def _tc2_body(deg_ref, acc_ref, h1p_ref, b1_ref, w2_ref, h2p_ref):
    dinv = _dinv(deg_ref)
    s = acc_ref[:NPAD, :] + acc_ref[NPAD:, :] + h1p_ref[...]
    z = jnp.maximum(dinv * s + b1_ref[...][None, :], 0.0)
    h2p_ref[...] = dinv * jnp.dot(z, w2_ref[...],
                                  preferred_element_type=jnp.float32)


def _tc3_body(deg_ref, acc_ref, h2p_ref, b2_ref, out_ref):
    dinv = _dinv(deg_ref)
    s = dinv * (acc_ref[:NPAD, :] + acc_ref[NPAD:, :] + h2p_ref[...])
    s = s + b2_ref[...][None, :]
    m = jnp.max(s, axis=1, keepdims=True)
    sh = s - m
    out_ref[...] = sh - jnp.log(jnp.sum(jnp.exp(sh), axis=1, keepdims=True))


def kernel(x, edge_index, W1, b1, W2, b2):
    e = edge_index.shape[1]
    ept = -(-e // (NW * K)) * K
    epad = ept * NW
    nch = ept // K

    src = edge_index[0].astype(jnp.int32)
    dst = edge_index[1].astype(jnp.int32)
    pad = jnp.full((epad - e,), N, jnp.int32)
    srcp = jnp.concatenate([src, pad])
    dstp = jnp.concatenate([dst, pad])
    xpad = jnp.pad(x, ((0, NPAD - N), (0, 0)))

    hid = W1.shape[1]
    out_ch = W2.shape[1]
    ones16 = jnp.ones((NPAD, 16), jnp.float32)
    z16 = jnp.zeros((NPAD, 16), jnp.float32)
    zh = jnp.zeros((NPAD, hid), jnp.float32)
    zo = jnp.zeros((NPAD, out_ch), jnp.float32)

    deg16 = _make_scatter(16, ept, nch)(ones16, srcp, dstp, z16)

    h1p = pl.pallas_call(
        _tc1_body,
        out_shape=jax.ShapeDtypeStruct((NPAD, hid), jnp.float32),
    )(deg16, xpad, W1)

    acc1 = _make_scatter(hid, ept, nch)(h1p, srcp, dstp, zh)

    h2p = pl.pallas_call(
        _tc2_body,
        out_shape=jax.ShapeDtypeStruct((NPAD, out_ch), jnp.float32),
    )(deg16, acc1, h1p, b1, W2)

    acc2 = _make_scatter(out_ch, ept, nch)(h2p, srcp, dstp, zo)

    outp = pl.pallas_call(
        _tc3_body,
        out_shape=jax.ShapeDtypeStruct((NPAD, out_ch), jnp.float32),
    )(deg16, acc2, h2p, b2)

    return outp[:N]


# speedup vs baseline: 10.7529x; 10.7529x over previous
import functools

import jax
import jax.numpy as jnp
from jax import lax
from jax.experimental import pallas as pl
from jax.experimental.pallas import tpu as pltpu
from jax.experimental.pallas import tpu_sc as plsc

N = 10000
NPAD = 10112
K = 128
NW = 32
NSUB = 16
RPT = NPAD // NSUB


def _scatter_body(ept, nch, h_hbm, src_hbm, dst_hbm, zero_hbm, out_hbm,
                  acc_sh, sidx, didx, rows, sem):
    cid = lax.axis_index("c")
    sid = lax.axis_index("s")
    wid = sid * 2 + cid

    @pl.when(sid == 0)
    def _zero():
        pltpu.sync_copy(zero_hbm, acc_sh)

    plsc.subcore_barrier()

    base = wid * ept

    def chunk(g, carry):
        eb = pl.multiple_of(base + g * K, 8)
        pltpu.sync_copy(src_hbm.at[pl.ds(eb, K)], sidx)
        pltpu.sync_copy(dst_hbm.at[pl.ds(eb, K)], didx)
        pltpu.async_copy(h_hbm.at[sidx], rows, sem).wait()
        pltpu.sync_copy(rows, acc_sh.at[didx], add=True)
        return carry

    lax.fori_loop(0, nch, chunk, 0)
    plsc.subcore_barrier()
    row0 = cid * NPAD + sid * RPT
    pltpu.sync_copy(acc_sh.at[pl.ds(sid * RPT, RPT)],
                    out_hbm.at[pl.ds(row0, RPT)])


@functools.lru_cache(maxsize=None)
def _make_scatter(c, ept, nch):
    mesh = plsc.VectorSubcoreMesh(core_axis_name="c", subcore_axis_name="s")
    return pl.kernel(
        functools.partial(_scatter_body, ept, nch),
        out_type=jax.ShapeDtypeStruct((2 * NPAD, c), jnp.float32),
        mesh=mesh,
        compiler_params=pltpu.CompilerParams(use_tc_tiling_on_sc=False),
        scratch_types=[
            pltpu.VMEM_SHARED((NPAD, c), jnp.float32),
            pltpu.VMEM((K,), jnp.int32),
            pltpu.VMEM((K,), jnp.int32),
            pltpu.VMEM((K, c), jnp.float32),
            pltpu.SemaphoreType.DMA,
        ],
    )


def _dinv(deg_ref):
    deg = deg_ref[:NPAD, 0:1] + deg_ref[NPAD:, 0:1] + 1.0
    row = lax.broadcasted_iota(jnp.int32, (NPAD, 1), 0)
    return jnp.where(row < N, lax.rsqrt(deg), 0.0)


def _tc1_body(deg_ref, x_ref, w1_ref, h1p_ref):
    h = jnp.dot(x_ref[...], w1_ref[...], preferred_element_type=jnp.float32)
    h1p_ref[...] = _dinv(deg_ref) * h


def _tc2_body(deg_ref, acc_ref, h1p_ref, b1_ref, w2_ref, h2p_ref):
    dinv = _dinv(deg_ref)
    s = acc_ref[:NPAD, :] + acc_ref[NPAD:, :] + h1p_ref[...]
    z = jnp.maximum(dinv * s + b1_ref[...][None, :], 0.0)
    h2p_ref[...] = dinv * jnp.dot(z, w2_ref[...],
                                  preferred_element_type=jnp.float32)


def _tc3_body(deg_ref, acc_ref, h2p_ref, b2_ref, out_ref):
    dinv = _dinv(deg_ref)
    s = dinv * (acc_ref[:NPAD, :] + acc_ref[NPAD:, :] + h2p_ref[...])
    s = s + b2_ref[...][None, :]
    m = jnp.max(s, axis=1, keepdims=True)
    sh = s - m
    out_ref[...] = sh - jnp.log(jnp.sum(jnp.exp(sh), axis=1, keepdims=True))


def kernel(x, edge_index, W1, b1, W2, b2):
    e = edge_index.shape[1]
    ept = -(-e // (NW * K)) * K
    epad = ept * NW
    nch = ept // K

    src = edge_index[0].astype(jnp.int32)
    dst = edge_index[1].astype(jnp.int32)
    pad = jnp.full((epad - e,), N, jnp.int32)
    srcp = jnp.concatenate([src, pad])
    dstp = jnp.concatenate([dst, pad])
    xpad = jnp.pad(x, ((0, NPAD - N), (0, 0)))

    hid = W1.shape[1]
    out_ch = W2.shape[1]
    ones16 = jnp.ones((NPAD, 16), jnp.float32)
    z16 = jnp.zeros((NPAD, 16), jnp.float32)
    zh = jnp.zeros((NPAD, hid), jnp.float32)
    zo = jnp.zeros((NPAD, out_ch), jnp.float32)

    deg16 = _make_scatter(16, ept, nch)(ones16, srcp, dstp, z16)

    h1p = pl.pallas_call(
        _tc1_body,
        out_shape=jax.ShapeDtypeStruct((NPAD, hid), jnp.float32),
    )(deg16, xpad, W1)

    acc1 = _make_scatter(hid, ept, nch)(h1p, srcp, dstp, zh)

    h2p = pl.pallas_call(
        _tc2_body,
        out_shape=jax.ShapeDtypeStruct((NPAD, out_ch), jnp.float32),
    )(deg16, acc1, h1p, b1, W2)

    acc2 = _make_scatter(out_ch, ept, nch)(h2p, srcp, dstp, zo)

    outp = pl.pallas_call(
        _tc3_body,
        out_shape=jax.ShapeDtypeStruct((NPAD, out_ch), jnp.float32),
    )(deg16, acc2, h2p, b2)

    return outp[:N]


# speedup vs baseline: 11.3958x; 1.0598x over previous
import functools

import jax
import jax.numpy as jnp
from jax import lax
from jax.experimental import pallas as pl
from jax.experimental.pallas import tpu as pltpu
from jax.experimental.pallas import tpu_sc as plsc

N = 10000
NPAD = 10112
K = 128
NW = 32
NSUB = 16
RPT = NPAD // NSUB
NBUF = 5
LEAD = 3


def _scatter_body(nch, h_hbm, src_hbm, dst_hbm, zero_hbm, out_hbm,
                  acc_sh, sidx, didx, rows, *sems):
    gsems = sems[:NBUF]
    ssems = sems[NBUF:]
    cid = lax.axis_index("c")
    sid = lax.axis_index("s")
    wid = sid * 2 + cid

    pltpu.sync_copy(src_hbm.at[pl.ds(wid * nch, nch)], sidx)
    pltpu.sync_copy(dst_hbm.at[pl.ds(wid * nch, nch)], didx)
    pltpu.sync_copy(zero_hbm.at[pl.ds(sid * RPT, RPT)],
                    acc_sh.at[pl.ds(sid * RPT, RPT)])
    for b in range(LEAD):
        pltpu.async_copy(h_hbm.at[sidx.at[b]], rows.at[b], gsems[b])
    plsc.subcore_barrier()

    lag = NBUF - LEAD

    def turn(i, carry):
        for b in range(NBUF):
            g = i * NBUF + b
            pltpu.make_async_copy(h_hbm.at[sidx.at[g]], rows.at[b],
                                  gsems[b]).wait()
            pltpu.async_copy(rows.at[b], acc_sh.at[didx.at[g]],
                             ssems[b], add=True)
            bn = (b + LEAD) % NBUF

            @pl.when(g >= lag)
            def _():
                pltpu.make_async_copy(rows.at[bn],
                                      acc_sh.at[didx.at[g - lag]],
                                      ssems[bn]).wait()

            @pl.when(g + LEAD < nch)
            def _():
                pltpu.async_copy(h_hbm.at[sidx.at[g + LEAD]], rows.at[bn],
                                 gsems[bn])
        return carry

    lax.fori_loop(0, nch // NBUF, turn, 0)
    for g in range(nch - lag, nch):
        pltpu.make_async_copy(rows.at[g % NBUF], acc_sh.at[didx.at[g]],
                              ssems[g % NBUF]).wait()
    plsc.subcore_barrier()
    row0 = cid * NPAD + sid * RPT
    pltpu.sync_copy(acc_sh.at[pl.ds(sid * RPT, RPT)],
                    out_hbm.at[pl.ds(row0, RPT)])


@functools.lru_cache(maxsize=None)
def _make_scatter(c, nch):
    mesh = plsc.VectorSubcoreMesh(core_axis_name="c", subcore_axis_name="s")
    return pl.kernel(
        functools.partial(_scatter_body, nch),
        out_type=jax.ShapeDtypeStruct((2 * NPAD, c), jnp.float32),
        mesh=mesh,
        compiler_params=pltpu.CompilerParams(use_tc_tiling_on_sc=False),
        scratch_types=[
            pltpu.VMEM_SHARED((NPAD, c), jnp.float32),
            pltpu.VMEM((nch, K), jnp.int32),
            pltpu.VMEM((nch, K), jnp.int32),
            pltpu.VMEM((NBUF, K, c), jnp.float32),
        ] + [pltpu.SemaphoreType.DMA] * (2 * NBUF),
    )


def _deg_body(nch, ones_hbm, dst_hbm, zero_hbm, out_hbm,
              acc_sh, didx, ones_v, ssem):
    cid = lax.axis_index("c")
    sid = lax.axis_index("s")
    wid = sid * 2 + cid

    pltpu.sync_copy(dst_hbm.at[pl.ds(wid * nch, nch)], didx)
    pltpu.sync_copy(ones_hbm, ones_v)
    pltpu.sync_copy(zero_hbm.at[pl.ds(sid * RPT, RPT)],
                    acc_sh.at[pl.ds(sid * RPT, RPT)])
    plsc.subcore_barrier()

    def fire(g, carry):
        pltpu.async_copy(ones_v, acc_sh.at[didx.at[g]], ssem, add=True)
        return carry

    lax.fori_loop(0, nch, fire, 0)

    def drain(g, carry):
        pltpu.make_async_copy(ones_v, acc_sh.at[didx.at[g]], ssem).wait()
        return carry

    lax.fori_loop(0, nch, drain, 0)
    plsc.subcore_barrier()
    row0 = cid * NPAD + sid * RPT
    pltpu.sync_copy(acc_sh.at[pl.ds(sid * RPT, RPT)],
                    out_hbm.at[pl.ds(row0, RPT)])


@functools.lru_cache(maxsize=None)
def _make_deg(nch):
    mesh = plsc.VectorSubcoreMesh(core_axis_name="c", subcore_axis_name="s")
    return pl.kernel(
        functools.partial(_deg_body, nch),
        out_type=jax.ShapeDtypeStruct((2 * NPAD, 16), jnp.float32),
        mesh=mesh,
        compiler_params=pltpu.CompilerParams(use_tc_tiling_on_sc=False),
        scratch_types=[
            pltpu.VMEM_SHARED((NPAD, 16), jnp.float32),
            pltpu.VMEM((nch, K), jnp.int32),
            pltpu.VMEM((K, 16), jnp.float32),
            pltpu.SemaphoreType.DMA,
        ],
    )


def _dinv(deg_ref):
    deg = deg_ref[:NPAD, 0:1] + deg_ref[NPAD:, 0:1] + 1.0
    row = lax.broadcasted_iota(jnp.int32, (NPAD, 1), 0)
    return jnp.where(row < N, lax.rsqrt(deg), 0.0)


def _tc1_body(deg_ref, x_ref, w1_ref, h1a_ref, h1b_ref):
    h = jnp.dot(x_ref[...], w1_ref[...], preferred_element_type=jnp.float32)
    dinv = _dinv(deg_ref)
    c = h.shape[1] // 2
    h1a_ref[...] = dinv * h[:, :c]
    h1b_ref[...] = dinv * h[:, c:]


def _tc2_body(deg_ref, acc_a_ref, acc_b_ref, h1a_ref, h1b_ref, b1_ref,
              w2_ref, h2p_ref):
    dinv = _dinv(deg_ref)
    c = h1a_ref.shape[1]
    sa = acc_a_ref[:NPAD, :] + acc_a_ref[NPAD:, :] + h1a_ref[...]
    sb = acc_b_ref[:NPAD, :] + acc_b_ref[NPAD:, :] + h1b_ref[...]
    za = jnp.maximum(dinv * sa + b1_ref[...][None, :c], 0.0)
    zb = jnp.maximum(dinv * sb + b1_ref[...][None, c:], 0.0)
    h2 = (jnp.dot(za, w2_ref[:c, :], preferred_element_type=jnp.float32)
          + jnp.dot(zb, w2_ref[c:, :], preferred_element_type=jnp.float32))
    h2p_ref[...] = dinv * h2


def _tc3_body(deg_ref, acc_ref, h2p_ref, b2_ref, out_ref):
    dinv = _dinv(deg_ref)
    s = dinv * (acc_ref[:NPAD, :] + acc_ref[NPAD:, :] + h2p_ref[...])
    s = s + b2_ref[...][None, :]
    m = jnp.max(s, axis=1, keepdims=True)
    sh = s - m
    out_ref[...] = sh - jnp.log(jnp.sum(jnp.exp(sh), axis=1, keepdims=True))


def kernel(x, edge_index, W1, b1, W2, b2):
    e = edge_index.shape[1]
    nch = -(-e // (NW * K * NBUF)) * NBUF
    ept = nch * K
    epad = ept * NW

    src = edge_index[0].astype(jnp.int32)
    dst = edge_index[1].astype(jnp.int32)
    pad = jnp.full((epad - e,), N, jnp.int32)
    srcp = jnp.concatenate([src, pad]).reshape(NW * nch, K)
    dstp = jnp.concatenate([dst, pad]).reshape(NW * nch, K)
    xpad = jnp.pad(x, ((0, NPAD - N), (0, 0)))

    hid = W1.shape[1]
    half = hid // 2
    out_ch = W2.shape[1]
    ones_k = jnp.ones((K, 16), jnp.float32)
    z16 = jnp.zeros((NPAD, 16), jnp.float32)
    zhalf = jnp.zeros((NPAD, half), jnp.float32)
    zo = jnp.zeros((NPAD, out_ch), jnp.float32)

    deg16 = _make_deg(nch)(ones_k, dstp, z16)

    h1a, h1b = pl.pallas_call(
        _tc1_body,
        out_shape=[jax.ShapeDtypeStruct((NPAD, half), jnp.float32),
                   jax.ShapeDtypeStruct((NPAD, half), jnp.float32)],
    )(deg16, xpad, W1)

    acc1a = _make_scatter(half, nch)(h1a, srcp, dstp, zhalf)
    acc1b = _make_scatter(half, nch)(h1b, srcp, dstp, zhalf)

    h2p = pl.pallas_call(
        _tc2_body,
        out_shape=jax.ShapeDtypeStruct((NPAD, out_ch), jnp.float32),
    )(deg16, acc1a, acc1b, h1a, h1b, b1, W2)

    acc2 = _make_scatter(out_ch, nch)(h2p, srcp, dstp, zo)

    outp = pl.pallas_call(
        _tc3_body,
        out_shape=jax.ShapeDtypeStruct((NPAD, out_ch), jnp.float32),
    )(deg16, acc2, h2p, b2)

    return outp[:N]
